# baseline (speedup 1.0000x reference)
import functools

import jax
import jax.numpy as jnp
from jax import lax
from jax.experimental import pallas as pl
from jax.experimental.pallas import tpu as pltpu
from jax.experimental.pallas import tpu_sc as plsc

NC = 2
NS = 16
NW = NC * NS
EB = 128
EPS = 1e-5


def _sc_gather(x, idx2):
    R, _ = idx2.shape
    n, c = x.shape
    nb = (R + NW - 1) // NW

    @functools.partial(
        pl.kernel,
        mesh=plsc.VectorSubcoreMesh(core_axis_name="c", subcore_axis_name="s"),
        out_type=jax.ShapeDtypeStruct((R * EB, c), x.dtype),
        scratch_types=[
            pltpu.VMEM((4, EB), jnp.int32),
            pltpu.VMEM((4, EB, c), x.dtype),
            pltpu.SemaphoreType.DMA,
            pltpu.SemaphoreType.DMA,
            [pltpu.SemaphoreType.DMA] * 4,
        ],
    )
    def k(x_hbm, idx_hbm, g_hbm, idx_v, rows_v, sem_i, sem_g, sem_s):
        ci = lax.axis_index("c")
        si = lax.axis_index("s")
        wid = si * NC + ci

        def r_of(t):
            return wid + t * NW

        def idx_cp(t, b):
            return pltpu.make_async_copy(idx_hbm.at[r_of(t)], idx_v.at[b], sem_i)

        def gat_cp(b):
            return pltpu.make_async_copy(
                x_hbm.at[idx_v.at[b]], rows_v.at[b], sem_g
            )

        def st_cp(t, b):
            return pltpu.make_async_copy(
                rows_v.at[b], g_hbm.at[pl.ds(r_of(t) * EB, EB)], sem_s[b]
            )

        for b in (0, 1):
            @pl.when(r_of(b) < R)
            def _():
                idx_cp(b, b).start()

        def round_(o, carry):
            for p in (0, 1):
                @pl.when((o % 2 == p) & (r_of(o * 2) < R))
                def _():
                    s0, s1 = 2 * p, 2 * p + 1
                    t0 = o * 2
                    t1 = t0 + 1
                    idx_cp(t0, s0).wait()

                    @pl.when(r_of(t1) < R)
                    def _():
                        idx_cp(t1, s1).wait()

                    @pl.when(t0 >= 4)
                    def _():
                        st_cp(t0 - 4, s0).wait()

                    @pl.when((t1 >= 4) & (r_of(t1 - 4) < R) & (r_of(t1) < R))
                    def _():
                        st_cp(t1 - 4, s1).wait()

                    gat_cp(s0).start()

                    @pl.when(r_of(t1) < R)
                    def _():
                        gat_cp(s1).start()

                    gat_cp(s0).wait()

                    @pl.when(r_of(t1) < R)
                    def _():
                        gat_cp(s1).wait()

                    o0, o1 = 2 * (1 - p), 2 * (1 - p) + 1

                    @pl.when(r_of(t0 + 2) < R)
                    def _():
                        idx_cp(t0 + 2, o0).start()

                    @pl.when(r_of(t1 + 2) < R)
                    def _():
                        idx_cp(t1 + 2, o1).start()

                    st_cp(t0, s0).start()

                    @pl.when(r_of(t1) < R)
                    def _():
                        st_cp(t1, s1).start()

            return carry

        nr = (nb + 1) // 2
        lax.fori_loop(0, nr, round_, 0)
        for t in range(max(0, nb - 5), nb):
            @pl.when((r_of(t) < R) & (r_of(t + 4) >= R))
            def _():
                st_cp(t, t % 4).wait()

    return k(x, idx2)


def _sc_scatter(y0, y1, col2, n):
    R, _ = col2.shape
    c = y0.shape[1]
    rh = R // NC
    nb = (rh + NS - 1) // NS
    sl = -(-n // (NS * 8)) * 8
    npad = sl * NS

    @functools.partial(
        pl.kernel,
        mesh=plsc.VectorSubcoreMesh(core_axis_name="c", subcore_axis_name="s"),
        out_type=(
            jax.ShapeDtypeStruct((NC * npad, c), jnp.float32),
            jax.ShapeDtypeStruct((NW, npad), jnp.float32),
        ),
        scratch_types=[
            pltpu.VMEM((4, EB), jnp.int32),
            pltpu.VMEM((2, EB, c), jnp.float32),
            pltpu.VMEM((npad,), jnp.float32),
            pltpu.VMEM_SHARED((npad, c), jnp.float32),
            [pltpu.SemaphoreType.DMA] * 4,
            pltpu.SemaphoreType.DMA,
            [pltpu.SemaphoreType.DMA] * 2,
        ],
        compiler_params=pltpu.CompilerParams(needs_layout_passes=False),
    )
    def k(y0_hbm, y1_hbm, col_hbm, sums_hbm, cnt_hbm, idx_v, y_v, cnt_v,
          acc_sh, sem_i, sem_y, sem_sc):
        ci = lax.axis_index("c")
        si = lax.axis_index("s")
        wid = si * NC + ci

        zeros16 = jnp.zeros((16,), jnp.float32)

        def zrow(rr, carry):
            for q in range(c // 16):
                y_v[0, rr, pl.ds(q * 16, 16)] = zeros16
            return carry

        lax.fori_loop(0, EB, zrow, 0)

        def zcnt(rr, carry):
            cnt_v[pl.ds(rr * 16, 16)] = zeros16
            return carry

        lax.fori_loop(0, npad // 16, zcnt, 0)
        for t in range((sl + EB - 1) // EB):
            w = min(EB, sl - t * EB)
            pltpu.sync_copy(
                y_v.at[0, pl.ds(0, w)],
                acc_sh.at[pl.ds(si * sl + t * EB, w)],
            )
        plsc.subcore_barrier()

        def valid(t):
            return si + t * NS < rh

        def r_of(t):
            return ci * rh + si + t * NS

        def idx_cp(t, b):
            return pltpu.make_async_copy(
                col_hbm.at[r_of(t)], idx_v.at[b], sem_i[b]
            )

        ones16 = jnp.ones((16,), jnp.float32)

        def pipeline(y_hbm):
            def y_cp(t, b):
                return pltpu.make_async_copy(
                    y_hbm.at[pl.ds((si + t * NS) * EB, EB)], y_v.at[b], sem_y
                )

            @pl.when(valid(0))
            def _():
                idx_cp(0, 0).start()
                y_cp(0, 0).start()

            @pl.when(valid(1))
            def _():
                idx_cp(1, 1).start()

            def round_(o, carry):
                for p in range(4):
                    @pl.when((o % 4 == p) & valid(o))
                    def _(p=p):
                        t = o - (o % 4) + p
                        bi = p % 4
                        by = p % 2
                        idx_cp(t, bi).wait()
                        y_cp(t, by).wait()
                        pltpu.async_copy(
                            y_v.at[by], acc_sh.at[idx_v.at[bi]], sem_sc[by],
                            add=True,
                        )
                        for q in range(EB // 16):
                            iv = idx_v[bi, pl.ds(q * 16, 16)]
                            plsc.addupdate_scatter(cnt_v, [iv], ones16)

                        @pl.when(t >= 1)
                        def _():
                            pltpu.make_async_copy(
                                y_v.at[1 - by],
                                acc_sh.at[idx_v.at[(p + 3) % 4]],
                                sem_sc[1 - by],
                            ).wait()

                        @pl.when(valid(t + 1))
                        def _():
                            y_cp(t + 1, 1 - by).start()

                        @pl.when(valid(t + 2))
                        def _():
                            idx_cp(t + 2, (p + 2) % 4).start()

                return carry

            lax.fori_loop(0, nb, round_, 0)
            for t in range(max(0, nb - 2), nb):
                @pl.when(valid(t) & ~valid(t + 1))
                def _():
                    pltpu.make_async_copy(
                        y_v.at[t % 2], acc_sh.at[idx_v.at[t % 4]],
                        sem_sc[t % 2],
                    ).wait()

        @pl.when(ci == 0)
        def _():
            pipeline(y0_hbm)

        @pl.when(ci == 1)
        def _():
            pipeline(y1_hbm)

        plsc.subcore_barrier()
        pltpu.sync_copy(
            acc_sh.at[pl.ds(si * sl, sl)],
            sums_hbm.at[pl.ds(ci * npad + si * sl, sl)],
        )
        pltpu.sync_copy(cnt_v, cnt_hbm.at[wid])

    sums, cnts = k(y0, y1, col2)
    return sums.reshape(NC, npad, c), cnts


def _tc_mlp_stats(a, bfeat, wa, wb, bias, blk, out_dtype=jnp.float32,
                  b_off=0):
    m, c = a.shape
    grid = m // blk

    def body(a_ref, b_ref, wa_ref, wb_ref, bias_ref, h_ref, s1_ref, s2_ref):
        i = pl.program_id(0)
        h = (
            jnp.dot(a_ref[:], wa_ref[:], preferred_element_type=jnp.float32)
            + jnp.dot(b_ref[:], wb_ref[:], preferred_element_type=jnp.float32)
            + bias_ref[:]
        )
        h_ref[:] = h.astype(out_dtype)

        @pl.when(i == 0)
        def _():
            s1_ref[:] = jnp.zeros_like(s1_ref)
            s2_ref[:] = jnp.zeros_like(s2_ref)

        s1_ref[:] += jnp.sum(h, axis=0, keepdims=True)
        s2_ref[:] += jnp.sum(h * h, axis=0, keepdims=True)

    return pl.pallas_call(
        body,
        grid=(grid,),
        in_specs=[
            pl.BlockSpec((blk, c), lambda i: (i, 0)),
            pl.BlockSpec((blk, c), lambda i: (i + b_off, 0)),
            pl.BlockSpec((c, c), lambda i: (0, 0)),
            pl.BlockSpec((c, c), lambda i: (0, 0)),
            pl.BlockSpec((1, c), lambda i: (0, 0)),
        ],
        out_specs=[
            pl.BlockSpec((blk, c), lambda i: (i, 0)),
            pl.BlockSpec((1, c), lambda i: (0, 0)),
            pl.BlockSpec((1, c), lambda i: (0, 0)),
        ],
        out_shape=[
            jax.ShapeDtypeStruct((m, c), out_dtype),
            jax.ShapeDtypeStruct((1, c), jnp.float32),
            jax.ShapeDtypeStruct((1, c), jnp.float32),
        ],
    )(a, bfeat, wa, wb, bias.reshape(1, c))


def _tc_node_mlp(x, parts, cnts, wa, wb, bias, blk):
    m, c = x.shape
    grid = m // blk

    def body(x_ref, p_ref, c_ref, wa_ref, wb_ref, bias_ref,
             h_ref, s1_ref, s2_ref):
        i = pl.program_id(0)
        p = p_ref[:]
        tot = p[0] + p[1]
        cnt = jnp.sum(c_ref[:], axis=1, keepdims=True)
        mean = tot / jnp.clip(cnt, 1.0, None)
        h = (
            jnp.dot(x_ref[:], wa_ref[:], preferred_element_type=jnp.float32)
            + jnp.dot(mean, wb_ref[:], preferred_element_type=jnp.float32)
            + bias_ref[:]
        )
        h_ref[:] = h

        @pl.when(i == 0)
        def _():
            s1_ref[:] = jnp.zeros_like(s1_ref)
            s2_ref[:] = jnp.zeros_like(s2_ref)

        s1_ref[:] += jnp.sum(h, axis=0, keepdims=True)
        s2_ref[:] += jnp.sum(h * h, axis=0, keepdims=True)

    return pl.pallas_call(
        body,
        grid=(grid,),
        in_specs=[
            pl.BlockSpec((blk, c), lambda i: (i, 0)),
            pl.BlockSpec((NC, blk, c), lambda i: (0, i, 0)),
            pl.BlockSpec((blk, NW), lambda i: (i, 0)),
            pl.BlockSpec((c, c), lambda i: (0, 0)),
            pl.BlockSpec((c, c), lambda i: (0, 0)),
            pl.BlockSpec((1, c), lambda i: (0, 0)),
        ],
        out_specs=[
            pl.BlockSpec((blk, c), lambda i: (i, 0)),
            pl.BlockSpec((1, c), lambda i: (0, 0)),
            pl.BlockSpec((1, c), lambda i: (0, 0)),
        ],
        out_shape=[
            jax.ShapeDtypeStruct((m, c), jnp.float32),
            jax.ShapeDtypeStruct((1, c), jnp.float32),
            jax.ShapeDtypeStruct((1, c), jnp.float32),
        ],
    )(x, parts, cnts, wa, wb, bias.reshape(1, c))


def _tc_bn_elu(z, s1, s2, gamma, beta, denom, blk):
    m, c = z.shape
    grid = m // blk

    def body(z_ref, s1_ref, s2_ref, g_ref, b_ref, y_ref):
        mean = s1_ref[:] * (1.0 / denom)
        var = s2_ref[:] * (1.0 / denom) - mean * mean
        scale = g_ref[:] * lax.rsqrt(var + EPS)
        shift = b_ref[:] - mean * scale
        t = z_ref[:].astype(jnp.float32) * scale + shift
        y_ref[:] = jnp.where(t > 0, t, jnp.exp(t) - 1.0)

    return pl.pallas_call(
        body,
        grid=(grid,),
        in_specs=[
            pl.BlockSpec((blk, c), lambda i: (i, 0)),
            pl.BlockSpec((1, c), lambda i: (0, 0)),
            pl.BlockSpec((1, c), lambda i: (0, 0)),
            pl.BlockSpec((1, c), lambda i: (0, 0)),
            pl.BlockSpec((1, c), lambda i: (0, 0)),
        ],
        out_specs=pl.BlockSpec((blk, c), lambda i: (i, 0)),
        out_shape=jax.ShapeDtypeStruct((m, c), jnp.float32),
    )(z, s1, s2, gamma.reshape(1, c), beta.reshape(1, c))


def kernel(x, edge_index, edge_attr, u, batch, W1, b1, g1, be1, W2, b2, g2, be2):
    n, c = x.shape
    e = edge_attr.shape[0]
    r = e // EB
    row2 = edge_index[0].reshape(r, EB)
    col2 = edge_index[1].reshape(r, EB)

    r2 = r // 2
    blk = 16000
    ga = _sc_gather(x, row2[:r2])
    gb = _sc_gather(x, row2[r2:])
    za, s1a, s2a = _tc_mlp_stats(
        ga, edge_attr, W1[:c], W1[c:], b1, blk=blk, out_dtype=jnp.bfloat16
    )
    zb, s1b, s2b = _tc_mlp_stats(
        gb, edge_attr, W1[:c], W1[c:], b1, blk=blk, out_dtype=jnp.bfloat16,
        b_off=(e // 2) // blk,
    )
    s1 = s1a + s1b
    s2 = s2a + s2b
    ya = _tc_bn_elu(za, s1, s2, g1, be1, denom=float(e), blk=blk)
    yb = _tc_bn_elu(zb, s1, s2, g1, be1, denom=float(e), blk=blk)
    parts, cnts = _sc_scatter(ya, yb, col2, n)
    h, t1, t2 = _tc_node_mlp(
        x, parts, cnts.T, W2[:c], W2[c:], b2, blk=10000,
    )
    return _tc_bn_elu(h, t1, t2, g2, be2, denom=float(n), blk=10000)


# speedup vs baseline: 1.0060x; 1.0060x over previous
import functools

import jax
import jax.numpy as jnp
from jax import lax
from jax.experimental import pallas as pl
from jax.experimental.pallas import tpu as pltpu
from jax.experimental.pallas import tpu_sc as plsc

NC = 2
NS = 16
NW = NC * NS
EB = 128
EPS = 1e-5


def _sc_gather(x, idx2):
    R, _ = idx2.shape
    n, c = x.shape
    nb = (R + NW - 1) // NW

    @functools.partial(
        pl.kernel,
        mesh=plsc.VectorSubcoreMesh(core_axis_name="c", subcore_axis_name="s"),
        out_type=jax.ShapeDtypeStruct((R * EB, c), x.dtype),
        scratch_types=[
            pltpu.VMEM((4, EB), jnp.int32),
            pltpu.VMEM((4, EB, c), x.dtype),
            pltpu.SemaphoreType.DMA,
            pltpu.SemaphoreType.DMA,
            [pltpu.SemaphoreType.DMA] * 4,
        ],
    )
    def k(x_hbm, idx_hbm, g_hbm, idx_v, rows_v, sem_i, sem_g, sem_s):
        ci = lax.axis_index("c")
        si = lax.axis_index("s")
        wid = si * NC + ci

        def r_of(t):
            return wid + t * NW

        def idx_cp(t, b):
            return pltpu.make_async_copy(idx_hbm.at[r_of(t)], idx_v.at[b], sem_i)

        def gat_cp(b):
            return pltpu.make_async_copy(
                x_hbm.at[idx_v.at[b]], rows_v.at[b], sem_g
            )

        def st_cp(t, b):
            return pltpu.make_async_copy(
                rows_v.at[b], g_hbm.at[pl.ds(r_of(t) * EB, EB)], sem_s[b]
            )

        for b in (0, 1):
            @pl.when(r_of(b) < R)
            def _():
                idx_cp(b, b).start()

        def round_(o, carry):
            for p in (0, 1):
                @pl.when((o % 2 == p) & (r_of(o * 2) < R))
                def _():
                    s0, s1 = 2 * p, 2 * p + 1
                    t0 = o * 2
                    t1 = t0 + 1
                    idx_cp(t0, s0).wait()

                    @pl.when(r_of(t1) < R)
                    def _():
                        idx_cp(t1, s1).wait()

                    @pl.when(t0 >= 4)
                    def _():
                        st_cp(t0 - 4, s0).wait()

                    @pl.when((t1 >= 4) & (r_of(t1 - 4) < R) & (r_of(t1) < R))
                    def _():
                        st_cp(t1 - 4, s1).wait()

                    gat_cp(s0).start()

                    @pl.when(r_of(t1) < R)
                    def _():
                        gat_cp(s1).start()

                    gat_cp(s0).wait()

                    @pl.when(r_of(t1) < R)
                    def _():
                        gat_cp(s1).wait()

                    o0, o1 = 2 * (1 - p), 2 * (1 - p) + 1

                    @pl.when(r_of(t0 + 2) < R)
                    def _():
                        idx_cp(t0 + 2, o0).start()

                    @pl.when(r_of(t1 + 2) < R)
                    def _():
                        idx_cp(t1 + 2, o1).start()

                    st_cp(t0, s0).start()

                    @pl.when(r_of(t1) < R)
                    def _():
                        st_cp(t1, s1).start()

            return carry

        nr = (nb + 1) // 2
        lax.fori_loop(0, nr, round_, 0)
        for t in range(max(0, nb - 5), nb):
            @pl.when((r_of(t) < R) & (r_of(t + 4) >= R))
            def _():
                st_cp(t, t % 4).wait()

    return k(x, idx2)


def _sc_scatter(y, col2, n):
    R, _ = col2.shape
    c = y.shape[1]
    rh = R // NC
    nb = (rh + NS - 1) // NS
    sl = -(-n // (NS * 8)) * 8
    npad = sl * NS

    @functools.partial(
        pl.kernel,
        mesh=plsc.VectorSubcoreMesh(core_axis_name="c", subcore_axis_name="s"),
        out_type=(
            jax.ShapeDtypeStruct((NC * npad, c), jnp.float32),
            jax.ShapeDtypeStruct((NW, npad), jnp.float32),
        ),
        scratch_types=[
            pltpu.VMEM((4, EB), jnp.int32),
            pltpu.VMEM((2, EB, c), jnp.float32),
            pltpu.VMEM((npad,), jnp.float32),
            pltpu.VMEM_SHARED((npad, c), jnp.float32),
            [pltpu.SemaphoreType.DMA] * 4,
            pltpu.SemaphoreType.DMA,
            [pltpu.SemaphoreType.DMA] * 2,
        ],
        compiler_params=pltpu.CompilerParams(needs_layout_passes=False),
    )
    def k(y_hbm, col_hbm, sums_hbm, cnt_hbm, idx_v, y_v, cnt_v, acc_sh,
          sem_i, sem_y, sem_sc):
        ci = lax.axis_index("c")
        si = lax.axis_index("s")
        wid = si * NC + ci

        zeros16 = jnp.zeros((16,), jnp.float32)

        def zrow(rr, carry):
            for q in range(c // 16):
                y_v[0, rr, pl.ds(q * 16, 16)] = zeros16
            return carry

        lax.fori_loop(0, EB, zrow, 0)

        def zcnt(rr, carry):
            cnt_v[pl.ds(rr * 16, 16)] = zeros16
            return carry

        lax.fori_loop(0, npad // 16, zcnt, 0)
        for t in range((sl + EB - 1) // EB):
            w = min(EB, sl - t * EB)
            pltpu.sync_copy(
                y_v.at[0, pl.ds(0, w)],
                acc_sh.at[pl.ds(si * sl + t * EB, w)],
            )
        plsc.subcore_barrier()

        def valid(t):
            return si + t * NS < rh

        def r_of(t):
            return ci * rh + si + t * NS

        def idx_cp(t, b):
            return pltpu.make_async_copy(
                col_hbm.at[r_of(t)], idx_v.at[b], sem_i[b]
            )

        def y_cp(t, b):
            return pltpu.make_async_copy(
                y_hbm.at[pl.ds(r_of(t) * EB, EB)], y_v.at[b], sem_y
            )

        ones16 = jnp.ones((16,), jnp.float32)

        @pl.when(valid(0))
        def _():
            idx_cp(0, 0).start()
            y_cp(0, 0).start()

        @pl.when(valid(1))
        def _():
            idx_cp(1, 1).start()

        def round_(o, carry):
            for p in range(4):
                @pl.when((o % 4 == p) & valid(o))
                def _(p=p):
                    t = o - (o % 4) + p
                    bi = p % 4
                    by = p % 2
                    idx_cp(t, bi).wait()
                    y_cp(t, by).wait()
                    pltpu.async_copy(
                        y_v.at[by], acc_sh.at[idx_v.at[bi]], sem_sc[by],
                        add=True,
                    )
                    for q in range(EB // 16):
                        iv = idx_v[bi, pl.ds(q * 16, 16)]
                        plsc.addupdate_scatter(cnt_v, [iv], ones16)

                    @pl.when(t >= 1)
                    def _():
                        pltpu.make_async_copy(
                            y_v.at[1 - by],
                            acc_sh.at[idx_v.at[(p + 3) % 4]],
                            sem_sc[1 - by],
                        ).wait()

                    @pl.when(valid(t + 1))
                    def _():
                        y_cp(t + 1, 1 - by).start()

                    @pl.when(valid(t + 2))
                    def _():
                        idx_cp(t + 2, (p + 2) % 4).start()

            return carry

        lax.fori_loop(0, nb, round_, 0)
        for t in range(max(0, nb - 2), nb):
            @pl.when(valid(t) & ~valid(t + 1))
            def _():
                pltpu.make_async_copy(
                    y_v.at[t % 2], acc_sh.at[idx_v.at[t % 4]], sem_sc[t % 2]
                ).wait()

        plsc.subcore_barrier()
        pltpu.sync_copy(
            acc_sh.at[pl.ds(si * sl, sl)],
            sums_hbm.at[pl.ds(ci * npad + si * sl, sl)],
        )
        pltpu.sync_copy(cnt_v, cnt_hbm.at[wid])

    sums, cnts = k(y, col2)
    return sums.reshape(NC, npad, c), cnts


def _tc_mlp_stats(a, bfeat, wa, wb, bias, blk, out_dtype=jnp.float32):
    m, c = a.shape
    grid = m // blk

    def body(a_ref, b_ref, wa_ref, wb_ref, bias_ref, h_ref, s1_ref, s2_ref):
        i = pl.program_id(0)
        h = (
            jnp.dot(a_ref[:], wa_ref[:], preferred_element_type=jnp.float32)
            + jnp.dot(b_ref[:], wb_ref[:], preferred_element_type=jnp.float32)
            + bias_ref[:]
        )
        h_ref[:] = h.astype(out_dtype)

        @pl.when(i == 0)
        def _():
            s1_ref[:] = jnp.zeros_like(s1_ref)
            s2_ref[:] = jnp.zeros_like(s2_ref)

        s1_ref[:] += jnp.sum(h, axis=0, keepdims=True)
        s2_ref[:] += jnp.sum(h * h, axis=0, keepdims=True)

    return pl.pallas_call(
        body,
        grid=(grid,),
        in_specs=[
            pl.BlockSpec((blk, c), lambda i: (i, 0)),
            pl.BlockSpec((blk, c), lambda i: (i, 0)),
            pl.BlockSpec((c, c), lambda i: (0, 0)),
            pl.BlockSpec((c, c), lambda i: (0, 0)),
            pl.BlockSpec((1, c), lambda i: (0, 0)),
        ],
        out_specs=[
            pl.BlockSpec((blk, c), lambda i: (i, 0)),
            pl.BlockSpec((1, c), lambda i: (0, 0)),
            pl.BlockSpec((1, c), lambda i: (0, 0)),
        ],
        out_shape=[
            jax.ShapeDtypeStruct((m, c), out_dtype),
            jax.ShapeDtypeStruct((1, c), jnp.float32),
            jax.ShapeDtypeStruct((1, c), jnp.float32),
        ],
    )(a, bfeat, wa, wb, bias.reshape(1, c))


def _tc_node_mlp(x, parts, cnts, wa, wb, bias, blk):
    m, c = x.shape
    grid = m // blk

    def body(x_ref, p_ref, c_ref, wa_ref, wb_ref, bias_ref,
             h_ref, s1_ref, s2_ref):
        i = pl.program_id(0)
        p = p_ref[:]
        tot = p[0] + p[1]
        cnt = jnp.sum(c_ref[:], axis=1, keepdims=True)
        mean = tot / jnp.clip(cnt, 1.0, None)
        h = (
            jnp.dot(x_ref[:], wa_ref[:], preferred_element_type=jnp.float32)
            + jnp.dot(mean, wb_ref[:], preferred_element_type=jnp.float32)
            + bias_ref[:]
        )
        h_ref[:] = h

        @pl.when(i == 0)
        def _():
            s1_ref[:] = jnp.zeros_like(s1_ref)
            s2_ref[:] = jnp.zeros_like(s2_ref)

        s1_ref[:] += jnp.sum(h, axis=0, keepdims=True)
        s2_ref[:] += jnp.sum(h * h, axis=0, keepdims=True)

    return pl.pallas_call(
        body,
        grid=(grid,),
        in_specs=[
            pl.BlockSpec((blk, c), lambda i: (i, 0)),
            pl.BlockSpec((NC, blk, c), lambda i: (0, i, 0)),
            pl.BlockSpec((blk, NW), lambda i: (i, 0)),
            pl.BlockSpec((c, c), lambda i: (0, 0)),
            pl.BlockSpec((c, c), lambda i: (0, 0)),
            pl.BlockSpec((1, c), lambda i: (0, 0)),
        ],
        out_specs=[
            pl.BlockSpec((blk, c), lambda i: (i, 0)),
            pl.BlockSpec((1, c), lambda i: (0, 0)),
            pl.BlockSpec((1, c), lambda i: (0, 0)),
        ],
        out_shape=[
            jax.ShapeDtypeStruct((m, c), jnp.float32),
            jax.ShapeDtypeStruct((1, c), jnp.float32),
            jax.ShapeDtypeStruct((1, c), jnp.float32),
        ],
    )(x, parts, cnts, wa, wb, bias.reshape(1, c))


def _tc_bn_elu(z, s1, s2, gamma, beta, denom, blk):
    m, c = z.shape
    grid = m // blk

    def body(z_ref, s1_ref, s2_ref, g_ref, b_ref, y_ref):
        mean = s1_ref[:] * (1.0 / denom)
        var = s2_ref[:] * (1.0 / denom) - mean * mean
        scale = g_ref[:] * lax.rsqrt(var + EPS)
        shift = b_ref[:] - mean * scale
        t = z_ref[:].astype(jnp.float32) * scale + shift
        y_ref[:] = jnp.where(t > 0, t, jnp.exp(t) - 1.0)

    return pl.pallas_call(
        body,
        grid=(grid,),
        in_specs=[
            pl.BlockSpec((blk, c), lambda i: (i, 0)),
            pl.BlockSpec((1, c), lambda i: (0, 0)),
            pl.BlockSpec((1, c), lambda i: (0, 0)),
            pl.BlockSpec((1, c), lambda i: (0, 0)),
            pl.BlockSpec((1, c), lambda i: (0, 0)),
        ],
        out_specs=pl.BlockSpec((blk, c), lambda i: (i, 0)),
        out_shape=jax.ShapeDtypeStruct((m, c), jnp.float32),
    )(z, s1, s2, gamma.reshape(1, c), beta.reshape(1, c))


def kernel(x, edge_index, edge_attr, u, batch, W1, b1, g1, be1, W2, b2, g2, be2):
    n, c = x.shape
    e = edge_attr.shape[0]
    r = e // EB
    row2 = edge_index[0].reshape(r, EB)
    col2 = edge_index[1].reshape(r, EB)

    g = _sc_gather(x, row2)
    z, s1, s2 = _tc_mlp_stats(
        g, edge_attr, W1[:c], W1[c:], b1, blk=16000, out_dtype=jnp.bfloat16
    )
    y = _tc_bn_elu(z, s1, s2, g1, be1, denom=float(e), blk=16000)
    parts, cnts = _sc_scatter(y, col2, n)
    h, t1, t2 = _tc_node_mlp(
        x, parts, cnts.T, W2[:c], W2[c:], b2, blk=10000,
    )
    return _tc_bn_elu(h, t1, t2, g2, be2, denom=float(n), blk=10000)
